# Initial kernel scaffold; baseline (speedup 1.0000x reference)
import functools

import jax
import jax.numpy as jnp
from jax import lax
from jax.experimental import pallas as pl
from jax.experimental.pallas import tpu as pltpu
from jax.experimental.pallas import tpu_sc as plsc

B = 16384
NEG = 5
D = 64
NC = 2
NS = 16
L = 16
NW = NC * NS
BPW = B // NW
CH = 128
NCHUNK = BPW // CH
GP = CH // L


def _log_sigmoid(x):
    m = jnp.minimum(x, 0.0)
    t = jnp.exp(-jnp.abs(x))
    s = t / (t + 2.0)
    s2 = s * s
    p = jnp.float32(1.0 / 11.0)
    p = 1.0 / 9.0 + s2 * p
    p = 1.0 / 7.0 + s2 * p
    p = 1.0 / 5.0 + s2 * p
    p = 1.0 / 3.0 + s2 * p
    p = 1.0 + s2 * p
    return m - 2.0 * (s * p)


_mesh = plsc.VectorSubcoreMesh(core_axis_name="c", subcore_axis_name="s")


@functools.partial(
    pl.kernel,
    mesh=_mesh,
    out_type=jax.ShapeDtypeStruct((NW, L), jnp.float32),
    scratch_types=[
        pltpu.VMEM((BPW,), jnp.int32),
        pltpu.VMEM((BPW,), jnp.int32),
        pltpu.VMEM((BPW * NEG,), jnp.int32),
        pltpu.VMEM((CH, D), jnp.float32),
        pltpu.VMEM((CH, D), jnp.float32),
        pltpu.VMEM((CH * NEG, D), jnp.float32),
        pltpu.VMEM((L,), jnp.float32),
        pltpu.SemaphoreType.DMA,
    ],
)
def _sg_kernel(pos_w_hbm, pos_v_hbm, neg_hbm, w_hbm, v_hbm, out_hbm,
               iw_v, iv_v, in_v, wr, vr, nr, accv, sem):
    wid = lax.axis_index("s") * NC + lax.axis_index("c")
    base = wid * BPW
    pltpu.sync_copy(pos_w_hbm.at[pl.ds(base, BPW)], iw_v)
    pltpu.sync_copy(pos_v_hbm.at[pl.ds(base, BPW)], iv_v)
    pltpu.sync_copy(neg_hbm.at[pl.ds(base * NEG, BPW * NEG)], in_v)

    lanes = lax.iota(jnp.int32, L)
    zero = jnp.zeros((L,), jnp.float32)

    def chunk_body(ci, tot):
        copies = [
            pltpu.async_copy(w_hbm.at[iw_v.at[pl.ds(ci * CH, CH)]], wr, sem),
            pltpu.async_copy(v_hbm.at[iv_v.at[pl.ds(ci * CH, CH)]], vr, sem),
        ]
        for k in range(NEG):
            copies.append(pltpu.async_copy(
                v_hbm.at[in_v.at[pl.ds(ci * CH * NEG + k * CH, CH)]],
                nr.at[pl.ds(k * CH, CH)], sem))
        for c in copies:
            c.wait()

        def group_body(g, tot):
            rows = lanes + g * L
            nrows0 = rows * NEG

            def d_body(d, carry):
                accp, a0, a1, a2, a3, a4 = carry
                dcol = jnp.full((L,), d, jnp.int32)
                wv = plsc.load_gather(wr, [rows, dcol])
                vv = plsc.load_gather(vr, [rows, dcol])
                accp = accp + wv * vv
                accs = [a0, a1, a2, a3, a4]
                out = []
                for k in range(NEG):
                    nv = plsc.load_gather(nr, [nrows0 + k, dcol])
                    out.append(accs[k] + nv * wv)
                return (accp, out[0], out[1], out[2], out[3], out[4])

            accp, a0, a1, a2, a3, a4 = lax.fori_loop(
                0, D, d_body, (zero, zero, zero, zero, zero, zero))
            tot = tot + _log_sigmoid(accp)
            for ak in (a0, a1, a2, a3, a4):
                tot = tot + _log_sigmoid(-ak)
            return tot

        return lax.fori_loop(0, GP, group_body, tot)

    tot = lax.fori_loop(0, NCHUNK, chunk_body, zero)
    accv[...] = tot
    pltpu.sync_copy(accv, out_hbm.at[wid])


def kernel(pos_w, pos_v, neg_v, w_emb, v_emb):
    pos_w = pos_w.astype(jnp.int32)
    pos_v = pos_v.astype(jnp.int32)
    neg = neg_v.astype(jnp.int32).reshape(-1)
    parts = _sg_kernel(pos_w, pos_v, neg, w_emb, v_emb)
    return -jnp.sum(parts)


# speedup vs baseline: 1.5730x; 1.5730x over previous
import functools

import jax
import jax.numpy as jnp
from jax import lax
from jax.experimental import pallas as pl
from jax.experimental.pallas import tpu as pltpu
from jax.experimental.pallas import tpu_sc as plsc

B = 16384
NEG = 5
D = 64
NC = 2
NS = 16
L = 16
NW = NC * NS
BPW = B // NW
CH = 128
NCHUNK = BPW // CH
GP = CH // L


def _log_sigmoid(x):
    m = jnp.minimum(x, 0.0)
    t = jnp.exp(-jnp.abs(x))
    s = t / (t + 2.0)
    s2 = s * s
    p = jnp.float32(1.0 / 11.0)
    p = 1.0 / 9.0 + s2 * p
    p = 1.0 / 7.0 + s2 * p
    p = 1.0 / 5.0 + s2 * p
    p = 1.0 / 3.0 + s2 * p
    p = 1.0 + s2 * p
    return m - 2.0 * (s * p)


_mesh = plsc.VectorSubcoreMesh(core_axis_name="c", subcore_axis_name="s")


@functools.partial(
    pl.kernel,
    mesh=_mesh,
    compiler_params=pltpu.CompilerParams(
        needs_layout_passes=False, use_tc_tiling_on_sc=False),
    out_type=jax.ShapeDtypeStruct((NW, L), jnp.float32),
    scratch_types=[
        pltpu.VMEM((BPW,), jnp.int32),
        pltpu.VMEM((BPW,), jnp.int32),
        pltpu.VMEM((BPW * NEG,), jnp.int32),
        pltpu.VMEM((CH, D), jnp.float32),
        pltpu.VMEM((CH, D), jnp.float32),
        pltpu.VMEM((CH * NEG, D), jnp.float32),
        pltpu.VMEM((L,), jnp.float32),
        pltpu.SemaphoreType.DMA,
    ],
)
def _sg_kernel(pos_w_hbm, pos_v_hbm, neg_hbm, w_hbm, v_hbm, out_hbm,
               iw_v, iv_v, in_v, wr, vr, nr, accv, sem):
    wid = lax.axis_index("s") * NC + lax.axis_index("c")
    base = wid * BPW
    pltpu.sync_copy(pos_w_hbm.at[pl.ds(base, BPW)], iw_v)
    pltpu.sync_copy(pos_v_hbm.at[pl.ds(base, BPW)], iv_v)
    pltpu.sync_copy(neg_hbm.at[pl.ds(base * NEG, BPW * NEG)], in_v)

    lanes = lax.iota(jnp.int32, L)
    zero = jnp.zeros((L,), jnp.float32)

    def chunk_body(ci, tot):
        copies = [
            pltpu.async_copy(w_hbm.at[iw_v.at[pl.ds(ci * CH, CH)]], wr, sem),
            pltpu.async_copy(v_hbm.at[iv_v.at[pl.ds(ci * CH, CH)]], vr, sem),
        ]
        for k in range(NEG):
            copies.append(pltpu.async_copy(
                v_hbm.at[in_v.at[pl.ds(ci * CH * NEG + k * CH, CH)]],
                nr.at[pl.ds(k * CH, CH)], sem))
        for c in copies:
            c.wait()

        def group_body(g, tot):
            rows = lanes + g * L
            nrows0 = rows * NEG

            def d_body(d, carry):
                accp, a0, a1, a2, a3, a4 = carry
                dcol = jnp.full((L,), d, jnp.int32)
                wv = plsc.load_gather(wr, [rows, dcol])
                vv = plsc.load_gather(vr, [rows, dcol])
                accp = accp + wv * vv
                accs = [a0, a1, a2, a3, a4]
                out = []
                for k in range(NEG):
                    nv = plsc.load_gather(nr, [nrows0 + k, dcol])
                    out.append(accs[k] + nv * wv)
                return (accp, out[0], out[1], out[2], out[3], out[4])

            accp, a0, a1, a2, a3, a4 = lax.fori_loop(
                0, D, d_body, (zero, zero, zero, zero, zero, zero))
            tot = tot + _log_sigmoid(accp)
            for ak in (a0, a1, a2, a3, a4):
                tot = tot + _log_sigmoid(-ak)
            return tot

        return lax.fori_loop(0, GP, group_body, tot)

    tot = lax.fori_loop(0, NCHUNK, chunk_body, zero)
    accv[...] = tot
    pltpu.sync_copy(accv, out_hbm.at[wid])


def kernel(pos_w, pos_v, neg_v, w_emb, v_emb):
    pos_w = pos_w.astype(jnp.int32)
    pos_v = pos_v.astype(jnp.int32)
    neg = neg_v.astype(jnp.int32).reshape(-1)
    parts = _sg_kernel(pos_w, pos_v, neg, w_emb, v_emb)
    return -jnp.sum(parts)


# speedup vs baseline: 1.5732x; 1.0002x over previous
import functools

import jax
import jax.numpy as jnp
from jax import lax
from jax.experimental import pallas as pl
from jax.experimental.pallas import tpu as pltpu
from jax.experimental.pallas import tpu_sc as plsc

B = 16384
NEG = 5
D = 64
NC = 2
NS = 16
L = 16
NW = NC * NS
BPW = B // NW
CH = 128
NCHUNK = BPW // CH
GP = CH // L


def _log_sigmoid(x):
    m = jnp.minimum(x, 0.0)
    t = jnp.exp(-jnp.abs(x))
    s = t / (t + 2.0)
    s2 = s * s
    p = jnp.float32(1.0 / 11.0)
    p = 1.0 / 9.0 + s2 * p
    p = 1.0 / 7.0 + s2 * p
    p = 1.0 / 5.0 + s2 * p
    p = 1.0 / 3.0 + s2 * p
    p = 1.0 + s2 * p
    return m - 2.0 * (s * p)


_mesh = plsc.VectorSubcoreMesh(core_axis_name="c", subcore_axis_name="s")


@functools.partial(
    pl.kernel,
    mesh=_mesh,
    compiler_params=pltpu.CompilerParams(
        needs_layout_passes=False, use_tc_tiling_on_sc=False),
    out_type=jax.ShapeDtypeStruct((NW, L), jnp.float32),
    scratch_types=[
        pltpu.VMEM((BPW,), jnp.int32),
        pltpu.VMEM((BPW,), jnp.int32),
        pltpu.VMEM((BPW * NEG,), jnp.int32),
        pltpu.VMEM((CH, D), jnp.float32),
        pltpu.VMEM((CH, D), jnp.float32),
        pltpu.VMEM((CH * NEG, D), jnp.float32),
        pltpu.VMEM((L,), jnp.float32),
        pltpu.SemaphoreType.DMA,
    ],
)
def _sg_kernel(pos_w_hbm, pos_v_hbm, neg_hbm, w_hbm, v_hbm, out_hbm,
               iw_v, iv_v, in_v, wr, vr, nr, accv, sem):
    wid = lax.axis_index("s") * NC + lax.axis_index("c")
    base = wid * BPW
    pltpu.sync_copy(pos_w_hbm.at[pl.ds(base, BPW)], iw_v)
    pltpu.sync_copy(pos_v_hbm.at[pl.ds(base, BPW)], iv_v)
    for k in range(NEG):
        pltpu.sync_copy(neg_hbm.at[pl.ds(k * B + base, BPW)],
                        in_v.at[pl.ds(k * BPW, BPW)])

    lanes = lax.iota(jnp.int32, L)
    zero = jnp.zeros((L,), jnp.float32)

    def chunk_body(ci, tot):
        copies = [
            pltpu.async_copy(w_hbm.at[iw_v.at[pl.ds(ci * CH, CH)]], wr, sem),
            pltpu.async_copy(v_hbm.at[iv_v.at[pl.ds(ci * CH, CH)]], vr, sem),
        ]
        for k in range(NEG):
            copies.append(pltpu.async_copy(
                v_hbm.at[in_v.at[pl.ds(k * BPW + ci * CH, CH)]],
                nr.at[pl.ds(k * CH, CH)], sem))
        for c in copies:
            c.wait()

        def group_body(g, tot):
            rows = lanes + g * L

            def d_body(d, carry):
                accp, a0, a1, a2, a3, a4 = carry
                dcol = jnp.full((L,), d, jnp.int32)
                wv = plsc.load_gather(wr, [rows, dcol])
                vv = plsc.load_gather(vr, [rows, dcol])
                accp = accp + wv * vv
                accs = [a0, a1, a2, a3, a4]
                out = []
                for k in range(NEG):
                    nv = plsc.load_gather(nr, [rows + k * CH, dcol])
                    out.append(accs[k] + nv * wv)
                return (accp, out[0], out[1], out[2], out[3], out[4])

            accp, a0, a1, a2, a3, a4 = lax.fori_loop(
                0, D, d_body, (zero, zero, zero, zero, zero, zero))
            tot = tot + _log_sigmoid(accp)
            for ak in (a0, a1, a2, a3, a4):
                tot = tot + _log_sigmoid(-ak)
            return tot

        return lax.fori_loop(0, GP, group_body, tot)

    tot = lax.fori_loop(0, NCHUNK, chunk_body, zero)
    accv[...] = tot
    pltpu.sync_copy(accv, out_hbm.at[wid])


def kernel(pos_w, pos_v, neg_v, w_emb, v_emb):
    pos_w = pos_w.astype(jnp.int32)
    pos_v = pos_v.astype(jnp.int32)
    neg = neg_v.astype(jnp.int32).T.reshape(-1)
    parts = _sg_kernel(pos_w, pos_v, neg, w_emb, v_emb)
    return -jnp.sum(parts)


# speedup vs baseline: 1.5754x; 1.0014x over previous
import functools

import jax
import jax.numpy as jnp
from jax import lax
from jax.experimental import pallas as pl
from jax.experimental.pallas import tpu as pltpu
from jax.experimental.pallas import tpu_sc as plsc

B = 16384
NEG = 5
D = 64
NC = 2
NS = 16
L = 16
NW = NC * NS
BPW = B // NW
CH = 128
NCHUNK = BPW // CH
GP = CH // L


def _log_sigmoid(x):
    m = jnp.minimum(x, 0.0)
    t = jnp.exp(-jnp.abs(x))
    s = t / (t + 2.0)
    s2 = s * s
    p = jnp.float32(1.0 / 11.0)
    p = 1.0 / 9.0 + s2 * p
    p = 1.0 / 7.0 + s2 * p
    p = 1.0 / 5.0 + s2 * p
    p = 1.0 / 3.0 + s2 * p
    p = 1.0 + s2 * p
    return m - 2.0 * (s * p)


_mesh = plsc.VectorSubcoreMesh(core_axis_name="c", subcore_axis_name="s")


@functools.partial(
    pl.kernel,
    mesh=_mesh,
    compiler_params=pltpu.CompilerParams(
        needs_layout_passes=False, use_tc_tiling_on_sc=False),
    out_type=jax.ShapeDtypeStruct((NW, L), jnp.float32),
    scratch_types=[
        pltpu.VMEM((BPW,), jnp.int32),
        pltpu.VMEM((BPW,), jnp.int32),
        pltpu.VMEM((BPW * NEG,), jnp.int32),
        pltpu.VMEM((CH, D), jnp.float32),
        pltpu.VMEM((CH, D), jnp.float32),
        pltpu.VMEM((CH * NEG, D), jnp.float32),
        pltpu.VMEM((L,), jnp.float32),
        pltpu.SemaphoreType.DMA,
    ],
)
def _sg_kernel(pos_w_hbm, pos_v_hbm, n0, n1, n2, n3, n4, w_hbm, v_hbm,
               out_hbm, iw_v, iv_v, in_v, wr, vr, nr, accv, sem):
    wid = lax.axis_index("s") * NC + lax.axis_index("c")
    base = wid * BPW
    pltpu.sync_copy(pos_w_hbm.at[pl.ds(base, BPW)], iw_v)
    pltpu.sync_copy(pos_v_hbm.at[pl.ds(base, BPW)], iv_v)
    for k, nk in enumerate((n0, n1, n2, n3, n4)):
        pltpu.sync_copy(nk.at[pl.ds(base, BPW)],
                        in_v.at[pl.ds(k * BPW, BPW)])

    lanes = lax.iota(jnp.int32, L)
    zero = jnp.zeros((L,), jnp.float32)

    def chunk_body(ci, tot):
        copies = [
            pltpu.async_copy(w_hbm.at[iw_v.at[pl.ds(ci * CH, CH)]], wr, sem),
            pltpu.async_copy(v_hbm.at[iv_v.at[pl.ds(ci * CH, CH)]], vr, sem),
        ]
        for k in range(NEG):
            copies.append(pltpu.async_copy(
                v_hbm.at[in_v.at[pl.ds(k * BPW + ci * CH, CH)]],
                nr.at[pl.ds(k * CH, CH)], sem))
        for c in copies:
            c.wait()

        def group_body(g, tot):
            rows = lanes + g * L

            def d_body(d, carry):
                accp, a0, a1, a2, a3, a4 = carry
                dcol = jnp.full((L,), d, jnp.int32)
                wv = plsc.load_gather(wr, [rows, dcol])
                vv = plsc.load_gather(vr, [rows, dcol])
                accp = accp + wv * vv
                accs = [a0, a1, a2, a3, a4]
                out = []
                for k in range(NEG):
                    nv = plsc.load_gather(nr, [rows + k * CH, dcol])
                    out.append(accs[k] + nv * wv)
                return (accp, out[0], out[1], out[2], out[3], out[4])

            accp, a0, a1, a2, a3, a4 = lax.fori_loop(
                0, D, d_body, (zero, zero, zero, zero, zero, zero))
            tot = tot + _log_sigmoid(accp)
            for ak in (a0, a1, a2, a3, a4):
                tot = tot + _log_sigmoid(-ak)
            return tot

        return lax.fori_loop(0, GP, group_body, tot)

    tot = lax.fori_loop(0, NCHUNK, chunk_body, zero)
    accv[...] = tot
    pltpu.sync_copy(accv, out_hbm.at[wid])


def kernel(pos_w, pos_v, neg_v, w_emb, v_emb):
    pos_w = pos_w.astype(jnp.int32)
    pos_v = pos_v.astype(jnp.int32)
    negs = [neg_v[:, k].astype(jnp.int32) for k in range(NEG)]
    parts = _sg_kernel(pos_w, pos_v, *negs, w_emb, v_emb)
    return -jnp.sum(parts)
